# baseline (speedup 1.0000x reference)
import functools

import jax
import jax.numpy as jnp
from jax import lax
from jax.experimental import pallas as pl
from jax.experimental.pallas import tpu as pltpu
from jax.experimental.pallas import tpu_sc as plsc

DIM = 64
LIST_LEN = 200
HALF = 100
NUM_LISTS = 7
LANES = 16


def _sc_encoder(ids, svec, w1p, w2t, b2, table):
    mesh = plsc.VectorSubcoreMesh(
        core_axis_name="c", subcore_axis_name="s",
        num_cores=2, num_subcores=16)

    @functools.partial(
        pl.kernel,
        mesh=mesh,
        out_type=jax.ShapeDtypeStruct((NUM_LISTS + 1, DIM), jnp.float32),
        compiler_params=pltpu.CompilerParams(
            needs_layout_passes=False, use_tc_tiling_on_sc=False),
        scratch_types=[
            pltpu.VMEM((2, HALF), jnp.int32),
            pltpu.VMEM((HALF, DIM), jnp.float32),
            pltpu.VMEM((HALF, DIM), jnp.float32),
            pltpu.VMEM((DIM,), jnp.float32),
            pltpu.VMEM((32,), jnp.float32),
            pltpu.VMEM((8, 32), jnp.float32),
            pltpu.VMEM((32, DIM), jnp.float32),
            pltpu.VMEM((DIM,), jnp.float32),
            pltpu.VMEM((48,), jnp.float32),
            pltpu.SemaphoreType.DMA,
            pltpu.SemaphoreType.DMA,
        ],
    )
    def enc(ids_hbm, svec_hbm, w1_hbm, w2_hbm, b2_hbm, table_hbm, out_hbm,
            idx_v, rows_a, rows_b, outv, s_v, w1_v, w2_v, b2_v, h_v,
            sem_a, sem_b):
        wid = lax.axis_index("s") * 2 + lax.axis_index("c")

        @pl.when(wid < NUM_LISTS)
        def _lists():
            pltpu.sync_copy(ids_hbm.at[wid], idx_v)
            cp_a = pltpu.async_copy(table_hbm.at[idx_v.at[0]], rows_a, sem_a)
            cp_b = pltpu.async_copy(table_hbm.at[idx_v.at[1]], rows_b, sem_b)
            cp_a.wait()
            cp_b.wait()

            def body(i, accs):
                return tuple(
                    acc
                    + rows_a[i, pl.ds(LANES * c, LANES)]
                    + rows_b[i, pl.ds(LANES * c, LANES)]
                    for c, acc in enumerate(accs)
                )

            z = jnp.zeros((LANES,), jnp.float32)
            accs = lax.fori_loop(0, HALF, body, (z, z, z, z))
            scale = jnp.float32(1.0 / LIST_LEN)
            for c in range(4):
                outv[pl.ds(LANES * c, LANES)] = accs[c] * scale
            pltpu.sync_copy(outv, out_hbm.at[wid])

        @pl.when(wid == NUM_LISTS)
        def _mlp():
            pltpu.sync_copy(svec_hbm, s_v)
            pltpu.sync_copy(w1_hbm, w1_v)
            pltpu.sync_copy(w2_hbm, w2_v)
            pltpu.sync_copy(b2_hbm, b2_v)

            def bcast(ref, j):
                idx = jnp.full((LANES,), j, jnp.int32)
                return plsc.load_gather(ref, [idx])

            hs = [jnp.zeros((LANES,), jnp.float32) for _ in range(2)]
            for k in range(8):
                sb = bcast(s_v, LANES + k)
                for c in range(2):
                    hs[c] = hs[c] + sb * w1_v[k, pl.ds(LANES * c, LANES)]
            for c in range(2):
                h_v[pl.ds(LANES * (c + 1), LANES)] = jnp.maximum(hs[c], 0.0)

            outs = [b2_v[pl.ds(LANES * c, LANES)] for c in range(4)]
            for j in range(32):
                hb = bcast(h_v, LANES + j)
                for c in range(4):
                    outs[c] = outs[c] + hb * w2_v[j, pl.ds(LANES * c, LANES)]
            for c in range(4):
                outv[pl.ds(LANES * c, LANES)] = outs[c]
            pltpu.sync_copy(outv, out_hbm.at[NUM_LISTS])

    return enc(ids, svec, w1p, w2t, b2, table)


def kernel(self_main_ids, self_field_ids, self_graveyard_ids, self_banish_ids,
           opp_fields_ids, opp_graveyard_ids, opp_banish_ids,
           self_banish_verso, opp_banish_verso, opp_fields_verso_card,
           phase_id, lp, adv_lp, opp_main,
           table, W1, b1, W2, b2):
    ids = jnp.stack([self_main_ids, self_field_ids, opp_fields_ids,
                     self_graveyard_ids, self_banish_ids,
                     opp_graveyard_ids, opp_banish_ids])
    ids = ids.astype(jnp.int32).reshape(NUM_LISTS, 2, HALF)

    scal = jnp.stack([phase_id, opp_main, lp, adv_lp,
                      opp_fields_verso_card, self_banish_verso,
                      opp_banish_verso]).astype(jnp.float32)
    svec = jnp.concatenate([jnp.zeros((16,), jnp.float32),
                            scal, jnp.ones((1,), jnp.float32),
                            jnp.zeros((8,), jnp.float32)])
    w1p = jnp.concatenate([W1.T, b1[None, :]], axis=0)
    w2t = W2.T

    out = _sc_encoder(ids, svec, w1p, w2t, b2, table)
    return out.reshape(NUM_LISTS * DIM + DIM)


# speedup vs baseline: 10.9701x; 10.9701x over previous
import functools

import jax
import jax.numpy as jnp
from jax import lax
from jax.experimental import pallas as pl
from jax.experimental.pallas import tpu as pltpu
from jax.experimental.pallas import tpu_sc as plsc

DIM = 64
LIST_LEN = 200
NUM_LISTS = 7
LANES = 16
IDS_PER_WORKER = 50
NUM_GATHER_WORKERS = 28
NBUF = 4


def _sc_gather(ids28, svec, w1p, w2t, b2, tt):
    mesh = plsc.VectorSubcoreMesh(
        core_axis_name="c", subcore_axis_name="s",
        num_cores=2, num_subcores=16)

    @functools.partial(
        pl.kernel,
        mesh=mesh,
        out_type=jax.ShapeDtypeStruct((32, 1, DIM), jnp.float32),
        compiler_params=pltpu.CompilerParams(needs_layout_passes=False),
        scratch_types=[
            pltpu.VMEM((1, 64), jnp.int32),
            pltpu.VMEM((DIM, 128), jnp.float32),
            pltpu.VMEM((DIM, 128), jnp.float32),
            pltpu.VMEM((DIM, 128), jnp.float32),
            pltpu.VMEM((DIM, 128), jnp.float32),
            pltpu.VMEM((1, DIM), jnp.float32),
            pltpu.VMEM((32,), jnp.float32),
            pltpu.VMEM((8, 32), jnp.float32),
            pltpu.VMEM((32, DIM), jnp.float32),
            pltpu.VMEM((DIM,), jnp.float32),
            pltpu.VMEM((48,), jnp.float32),
            pltpu.SemaphoreType.DMA,
            pltpu.SemaphoreType.DMA,
            pltpu.SemaphoreType.DMA,
            pltpu.SemaphoreType.DMA,
        ],
    )
    def enc(ids_hbm, svec_hbm, w1_hbm, w2_hbm, b2_hbm, tt_hbm, out_hbm,
            idx_v, buf0, buf1, buf2, buf3, outv,
            s_v, w1_v, w2_v, b2_v, h_v,
            sem0, sem1, sem2, sem3):
        wid = lax.axis_index("s") * 2 + lax.axis_index("c")
        bufs = [buf0, buf1, buf2, buf3]
        sems = [sem0, sem1, sem2, sem3]

        @pl.when(wid < NUM_GATHER_WORKERS)
        def _lists():
            pltpu.sync_copy(ids_hbm.at[wid], idx_v)
            ivs = [idx_v[0, pl.ds(LANES * g, LANES)] for g in range(4)]

            cps = [None] * IDS_PER_WORKER
            lanes = [None] * IDS_PER_WORKER

            def fire(k, slot):
                r = ivs[k // LANES][k % LANES]
                jt = pl.multiple_of((r >> 7) << 7, 128)
                lanes[k] = r & 127
                cps[k] = pltpu.async_copy(
                    tt_hbm.at[:, pl.ds(jt, 128)], bufs[slot], sems[slot])

            for k in range(NBUF):
                fire(k, k)
            z = jnp.zeros((LANES,), jnp.float32)
            accs = [z, z, z, z]
            iotas = [lax.iota(jnp.int32, LANES) + LANES * c for c in range(4)]
            for k in range(IDS_PER_WORKER):
                slot = k % NBUF
                cps[k].wait()
                colidx = jnp.full((LANES,), lanes[k], jnp.int32)
                for c in range(4):
                    rows = plsc.load_gather(bufs[slot], [iotas[c], colidx])
                    accs[c] = accs[c] + rows
                if k + NBUF < IDS_PER_WORKER:
                    fire(k + NBUF, slot)
            for c in range(4):
                outv[0, pl.ds(LANES * c, LANES)] = accs[c]
            pltpu.sync_copy(outv, out_hbm.at[wid])

        @pl.when(wid == NUM_GATHER_WORKERS)
        def _mlp():
            pltpu.sync_copy(svec_hbm, s_v)
            pltpu.sync_copy(w1_hbm, w1_v)
            pltpu.sync_copy(w2_hbm, w2_v)
            pltpu.sync_copy(b2_hbm, b2_v)

            def bcast(ref, j):
                idx = jnp.full((LANES,), j, jnp.int32)
                return plsc.load_gather(ref, [idx])

            hs = [jnp.zeros((LANES,), jnp.float32) for _ in range(2)]
            for k in range(8):
                sb = bcast(s_v, LANES + k)
                for c in range(2):
                    hs[c] = hs[c] + sb * w1_v[k, pl.ds(LANES * c, LANES)]
            for c in range(2):
                h_v[pl.ds(LANES * (c + 1), LANES)] = jnp.maximum(hs[c], 0.0)

            outs = [b2_v[pl.ds(LANES * c, LANES)] for c in range(4)]
            for j in range(32):
                hb = bcast(h_v, LANES + j)
                for c in range(4):
                    outs[c] = outs[c] + hb * w2_v[j, pl.ds(LANES * c, LANES)]
            for c in range(4):
                outv[0, pl.ds(LANES * c, LANES)] = outs[c]
            pltpu.sync_copy(outv, out_hbm.at[NUM_GATHER_WORKERS])

    return enc(ids28, svec, w1p, w2t, b2, tt)


def _tc_combine_body(x_ref, o_ref):
    scale = jnp.float32(1.0 / LIST_LEN)
    o_ref[pl.ds(0, NUM_LISTS), :] = (
        x_ref[pl.ds(0, NUM_LISTS), :]
        + x_ref[pl.ds(NUM_LISTS, NUM_LISTS), :]
        + x_ref[pl.ds(2 * NUM_LISTS, NUM_LISTS), :]
        + x_ref[pl.ds(3 * NUM_LISTS, NUM_LISTS), :]) * scale
    o_ref[pl.ds(NUM_LISTS, 1), :] = x_ref[pl.ds(NUM_GATHER_WORKERS, 1), :]


def _tc_combine(partials):
    return pl.pallas_call(
        _tc_combine_body,
        out_shape=jax.ShapeDtypeStruct((NUM_LISTS + 1, DIM), jnp.float32),
    )(partials)


def kernel(self_main_ids, self_field_ids, self_graveyard_ids, self_banish_ids,
           opp_fields_ids, opp_graveyard_ids, opp_banish_ids,
           self_banish_verso, opp_banish_verso, opp_fields_verso_card,
           phase_id, lp, adv_lp, opp_main,
           table, W1, b1, W2, b2):
    ids = jnp.stack([self_main_ids, self_field_ids, opp_fields_ids,
                     self_graveyard_ids, self_banish_ids,
                     opp_graveyard_ids, opp_banish_ids]).astype(jnp.int32)
    ids28 = ids.reshape(NUM_LISTS, 4, IDS_PER_WORKER)
    ids28 = ids28.transpose(1, 0, 2).reshape(NUM_GATHER_WORKERS,
                                             IDS_PER_WORKER)
    ids28 = jnp.pad(ids28, ((0, 0), (0, 64 - IDS_PER_WORKER)))
    ids28 = ids28.reshape(NUM_GATHER_WORKERS, 1, 64)

    scal = jnp.stack([phase_id, opp_main, lp, adv_lp,
                      opp_fields_verso_card, self_banish_verso,
                      opp_banish_verso]).astype(jnp.float32)
    svec = jnp.concatenate([jnp.zeros((16,), jnp.float32),
                            scal, jnp.ones((1,), jnp.float32),
                            jnp.zeros((8,), jnp.float32)])
    w1p = jnp.concatenate([W1.T, b1[None, :]], axis=0)
    w2t = W2.T

    partials = _sc_gather(ids28, svec, w1p, w2t, b2, table.T)
    out = _tc_combine(partials.reshape(32, DIM))
    return out.reshape(NUM_LISTS * DIM + DIM)


# speedup vs baseline: 13.1036x; 1.1945x over previous
import functools

import jax
import jax.numpy as jnp
from jax import lax
from jax.experimental import pallas as pl
from jax.experimental.pallas import tpu as pltpu
from jax.experimental.pallas import tpu_sc as plsc

DIM = 64
LIST_LEN = 200
NUM_LISTS = 7
LANES = 16
IDS_PER_WORKER = 50
NUM_GATHER_WORKERS = 28
NBUF = 6


def _sc_gather(id_lists, svec, w1p, w2t, b2, tt):
    mesh = plsc.VectorSubcoreMesh(
        core_axis_name="c", subcore_axis_name="s",
        num_cores=2, num_subcores=16)

    @functools.partial(
        pl.kernel,
        mesh=mesh,
        out_type=jax.ShapeDtypeStruct((32, 1, DIM), jnp.float32),
        compiler_params=pltpu.CompilerParams(needs_layout_passes=False),
        scratch_types=[
            pltpu.VMEM((LIST_LEN,), jnp.int32),
            *[pltpu.VMEM((DIM, 128), jnp.float32) for _ in range(NBUF)],
            pltpu.VMEM((1, DIM), jnp.float32),
            pltpu.VMEM((32,), jnp.float32),
            pltpu.VMEM((8, 32), jnp.float32),
            pltpu.VMEM((32, DIM), jnp.float32),
            pltpu.VMEM((DIM,), jnp.float32),
            pltpu.VMEM((48,), jnp.float32),
            *[pltpu.SemaphoreType.DMA for _ in range(NBUF)],
        ],
    )
    def enc(l0, l1, l2, l3, l4, l5, l6,
            svec_hbm, w1_hbm, w2_hbm, b2_hbm, tt_hbm, out_hbm,
            idx_v, *rest):
        bufs = list(rest[:NBUF])
        (outv, s_v, w1_v, w2_v, b2_v, h_v) = rest[NBUF:NBUF + 6]
        sems = list(rest[NBUF + 6:])
        wid = lax.axis_index("s") * 2 + lax.axis_index("c")

        @pl.when(wid < NUM_GATHER_WORKERS)
        def _lists():
            lst = wid // 4
            q = wid % 4
            for i, ref in enumerate((l0, l1, l2, l3, l4, l5, l6)):
                @pl.when(lst == i)
                def _(ref=ref):
                    pltpu.sync_copy(ref, idx_v)

            iota = lax.iota(jnp.int32, LANES)
            base = q * IDS_PER_WORKER
            ivs = []
            for g in range(4):
                gidx = iota + (base + LANES * g)
                if g == 3:
                    gidx = jnp.minimum(gidx, LIST_LEN - 1)
                ivs.append(plsc.load_gather(idx_v, [gidx]))

            cps = [None] * IDS_PER_WORKER
            lanes = [None] * IDS_PER_WORKER

            def fire(k, slot):
                r = ivs[k // LANES][k % LANES]
                jt = pl.multiple_of((r >> 7) << 7, 128)
                lanes[k] = r & 127
                cps[k] = pltpu.async_copy(
                    tt_hbm.at[:, pl.ds(jt, 128)], bufs[slot], sems[slot])

            for k in range(NBUF):
                fire(k, k)
            z = jnp.zeros((LANES,), jnp.float32)
            accs = [z, z, z, z]
            iotas = [iota + LANES * c for c in range(4)]
            for k in range(IDS_PER_WORKER):
                slot = k % NBUF
                cps[k].wait()
                colidx = jnp.full((LANES,), lanes[k], jnp.int32)
                for c in range(4):
                    rows = plsc.load_gather(bufs[slot], [iotas[c], colidx])
                    accs[c] = accs[c] + rows
                if k + NBUF < IDS_PER_WORKER:
                    fire(k + NBUF, slot)
            for c in range(4):
                outv[0, pl.ds(LANES * c, LANES)] = accs[c]
            pltpu.sync_copy(outv, out_hbm.at[NUM_LISTS * q + lst])

        @pl.when(wid == NUM_GATHER_WORKERS)
        def _mlp():
            pltpu.sync_copy(svec_hbm, s_v)
            pltpu.sync_copy(w1_hbm, w1_v)
            pltpu.sync_copy(w2_hbm, w2_v)
            pltpu.sync_copy(b2_hbm, b2_v)

            def bcast(ref, j):
                idx = jnp.full((LANES,), j, jnp.int32)
                return plsc.load_gather(ref, [idx])

            hs = [jnp.zeros((LANES,), jnp.float32) for _ in range(2)]
            for k in range(8):
                sb = bcast(s_v, LANES + k)
                for c in range(2):
                    hs[c] = hs[c] + sb * w1_v[k, pl.ds(LANES * c, LANES)]
            for c in range(2):
                h_v[pl.ds(LANES * (c + 1), LANES)] = jnp.maximum(hs[c], 0.0)

            outs = [b2_v[pl.ds(LANES * c, LANES)] for c in range(4)]
            for j in range(32):
                hb = bcast(h_v, LANES + j)
                for c in range(4):
                    outs[c] = outs[c] + hb * w2_v[j, pl.ds(LANES * c, LANES)]
            for c in range(4):
                outv[0, pl.ds(LANES * c, LANES)] = outs[c]
            pltpu.sync_copy(outv, out_hbm.at[NUM_GATHER_WORKERS])

    return enc(*id_lists, svec, w1p, w2t, b2, tt)


def _tc_combine_body(x_ref, o_ref):
    scale = jnp.float32(1.0 / LIST_LEN)
    o_ref[pl.ds(0, NUM_LISTS), :] = (
        x_ref[pl.ds(0, NUM_LISTS), 0, :]
        + x_ref[pl.ds(NUM_LISTS, NUM_LISTS), 0, :]
        + x_ref[pl.ds(2 * NUM_LISTS, NUM_LISTS), 0, :]
        + x_ref[pl.ds(3 * NUM_LISTS, NUM_LISTS), 0, :]) * scale
    o_ref[pl.ds(NUM_LISTS, 1), :] = x_ref[pl.ds(NUM_GATHER_WORKERS, 1), 0, :]


def _tc_combine(partials):
    return pl.pallas_call(
        _tc_combine_body,
        out_shape=jax.ShapeDtypeStruct((NUM_LISTS + 1, DIM), jnp.float32),
    )(partials)


def kernel(self_main_ids, self_field_ids, self_graveyard_ids, self_banish_ids,
           opp_fields_ids, opp_graveyard_ids, opp_banish_ids,
           self_banish_verso, opp_banish_verso, opp_fields_verso_card,
           phase_id, lp, adv_lp, opp_main,
           table, W1, b1, W2, b2):
    id_lists = [self_main_ids.astype(jnp.int32),
                self_field_ids.astype(jnp.int32),
                opp_fields_ids.astype(jnp.int32),
                self_graveyard_ids.astype(jnp.int32),
                self_banish_ids.astype(jnp.int32),
                opp_graveyard_ids.astype(jnp.int32),
                opp_banish_ids.astype(jnp.int32)]

    scal = jnp.stack([phase_id, opp_main, lp, adv_lp,
                      opp_fields_verso_card, self_banish_verso,
                      opp_banish_verso]).astype(jnp.float32)
    svec = jnp.concatenate([jnp.zeros((16,), jnp.float32),
                            scal, jnp.ones((1,), jnp.float32),
                            jnp.zeros((8,), jnp.float32)])
    w1p = jnp.concatenate([W1.T, b1[None, :]], axis=0)
    w2t = W2.T

    partials = _sc_gather(id_lists, svec, w1p, w2t, b2, table.T)
    out = _tc_combine(partials)
    return out.reshape(NUM_LISTS * DIM + DIM)


# speedup vs baseline: 13.2585x; 1.0118x over previous
import functools

import jax
import jax.numpy as jnp
from jax import lax
from jax.experimental import pallas as pl
from jax.experimental.pallas import tpu as pltpu
from jax.experimental.pallas import tpu_sc as plsc

DIM = 64
LIST_LEN = 200
NUM_LISTS = 7
LANES = 16
IDS_PER_WORKER = 50
NUM_GATHER_WORKERS = 28
NBUF = 6


def _sc_gather(id_lists, tt):
    mesh = plsc.VectorSubcoreMesh(
        core_axis_name="c", subcore_axis_name="s",
        num_cores=2, num_subcores=16)

    @functools.partial(
        pl.kernel,
        mesh=mesh,
        out_type=jax.ShapeDtypeStruct((NUM_GATHER_WORKERS, 1, DIM),
                                      jnp.float32),
        compiler_params=pltpu.CompilerParams(needs_layout_passes=False),
        scratch_types=[
            pltpu.VMEM((LIST_LEN,), jnp.int32),
            *[pltpu.VMEM((DIM, 128), jnp.float32) for _ in range(NBUF)],
            pltpu.VMEM((1, DIM), jnp.float32),
            *[pltpu.SemaphoreType.DMA for _ in range(NBUF)],
        ],
    )
    def enc(l0, l1, l2, l3, l4, l5, l6, tt_hbm, out_hbm, idx_v, *rest):
        bufs = list(rest[:NBUF])
        outv = rest[NBUF]
        sems = list(rest[NBUF + 1:])
        wid = lax.axis_index("s") * 2 + lax.axis_index("c")

        @pl.when(wid < NUM_GATHER_WORKERS)
        def _lists():
            lst = wid // 4
            q = wid % 4
            for i, ref in enumerate((l0, l1, l2, l3, l4, l5, l6)):
                @pl.when(lst == i)
                def _(ref=ref):
                    pltpu.sync_copy(ref, idx_v)

            iota = lax.iota(jnp.int32, LANES)
            base = q * IDS_PER_WORKER
            ivs = []
            for g in range(4):
                gidx = iota + (base + LANES * g)
                if g == 3:
                    gidx = jnp.minimum(gidx, LIST_LEN - 1)
                ivs.append(plsc.load_gather(idx_v, [gidx]))

            cps = [None] * IDS_PER_WORKER
            lanes = [None] * IDS_PER_WORKER

            def fire(k, slot):
                r = ivs[k // LANES][k % LANES]
                jt = pl.multiple_of((r >> 7) << 7, 128)
                lanes[k] = r & 127
                cps[k] = pltpu.async_copy(
                    tt_hbm.at[:, pl.ds(jt, 128)], bufs[slot], sems[slot])

            for k in range(NBUF):
                fire(k, k)
            z = jnp.zeros((LANES,), jnp.float32)
            accs = [z, z, z, z]
            iotas = [iota + LANES * c for c in range(4)]
            for k in range(IDS_PER_WORKER):
                slot = k % NBUF
                cps[k].wait()
                colidx = jnp.full((LANES,), lanes[k], jnp.int32)
                for c in range(4):
                    rows = plsc.load_gather(bufs[slot], [iotas[c], colidx])
                    accs[c] = accs[c] + rows
                if k + NBUF < IDS_PER_WORKER:
                    fire(k + NBUF, slot)
            for c in range(4):
                outv[0, pl.ds(LANES * c, LANES)] = accs[c]
            pltpu.sync_copy(outv, out_hbm.at[NUM_LISTS * q + lst])

    return enc(*id_lists, tt)


def _tc_combine_body(svec_ref, x_ref, w1_ref, b1_ref, w2_ref, b2_ref, o_ref):
    scale = jnp.float32(1.0 / LIST_LEN)
    o_ref[pl.ds(0, NUM_LISTS), :] = (
        x_ref[pl.ds(0, NUM_LISTS), 0, :]
        + x_ref[pl.ds(NUM_LISTS, NUM_LISTS), 0, :]
        + x_ref[pl.ds(2 * NUM_LISTS, NUM_LISTS), 0, :]
        + x_ref[pl.ds(3 * NUM_LISTS, NUM_LISTS), 0, :]) * scale
    w1 = w1_ref[...]
    h = b1_ref[...]
    for k in range(NUM_LISTS):
        h = h + svec_ref[k] * w1[:, k]
    h = jnp.maximum(h, 0.0)
    w2 = w2_ref[...]
    out = b2_ref[...]
    for j in range(32):
        out = out + h[j] * w2[:, j]
    o_ref[pl.ds(NUM_LISTS, 1), :] = out[None, :]


def _tc_combine(svec, partials, W1, b1, W2, b2):
    return pl.pallas_call(
        _tc_combine_body,
        in_specs=[
            pl.BlockSpec(memory_space=pltpu.SMEM),
            pl.BlockSpec(memory_space=pltpu.VMEM),
            pl.BlockSpec(memory_space=pltpu.VMEM),
            pl.BlockSpec(memory_space=pltpu.VMEM),
            pl.BlockSpec(memory_space=pltpu.VMEM),
            pl.BlockSpec(memory_space=pltpu.VMEM),
        ],
        out_shape=jax.ShapeDtypeStruct((NUM_LISTS + 1, DIM), jnp.float32),
    )(svec, partials, W1, b1, W2, b2)


def kernel(self_main_ids, self_field_ids, self_graveyard_ids, self_banish_ids,
           opp_fields_ids, opp_graveyard_ids, opp_banish_ids,
           self_banish_verso, opp_banish_verso, opp_fields_verso_card,
           phase_id, lp, adv_lp, opp_main,
           table, W1, b1, W2, b2):
    id_lists = [self_main_ids.astype(jnp.int32),
                self_field_ids.astype(jnp.int32),
                opp_fields_ids.astype(jnp.int32),
                self_graveyard_ids.astype(jnp.int32),
                self_banish_ids.astype(jnp.int32),
                opp_graveyard_ids.astype(jnp.int32),
                opp_banish_ids.astype(jnp.int32)]
    svec = jnp.stack([phase_id, opp_main, lp, adv_lp,
                      opp_fields_verso_card, self_banish_verso,
                      opp_banish_verso]).astype(jnp.float32)

    partials = _sc_gather(id_lists, table.T)
    out = _tc_combine(svec, partials, W1, b1, W2, b2)
    return out.reshape(NUM_LISTS * DIM + DIM)


# speedup vs baseline: 13.6402x; 1.0288x over previous
import functools

import jax
import jax.numpy as jnp
from jax import lax
from jax.experimental import pallas as pl
from jax.experimental.pallas import tpu as pltpu
from jax.experimental.pallas import tpu_sc as plsc

DIM = 64
LIST_LEN = 200
NUM_LISTS = 7
LANES = 16
IDS_PER_WORKER = 50
NUM_GATHER_WORKERS = 28
NBUF = 10


def _sc_gather(id_lists, tt):
    mesh = plsc.VectorSubcoreMesh(
        core_axis_name="c", subcore_axis_name="s",
        num_cores=2, num_subcores=16)

    @functools.partial(
        pl.kernel,
        mesh=mesh,
        out_type=jax.ShapeDtypeStruct((NUM_GATHER_WORKERS, 1, DIM),
                                      jnp.float32),
        compiler_params=pltpu.CompilerParams(needs_layout_passes=False),
        scratch_types=[
            pltpu.VMEM((LIST_LEN,), jnp.int32),
            *[pltpu.VMEM((DIM, 128), jnp.float32) for _ in range(NBUF)],
            pltpu.VMEM((1, DIM), jnp.float32),
            *[pltpu.SemaphoreType.DMA for _ in range(NBUF)],
        ],
    )
    def enc(l0, l1, l2, l3, l4, l5, l6, tt_hbm, out_hbm, idx_v, *rest):
        bufs = list(rest[:NBUF])
        outv = rest[NBUF]
        sems = list(rest[NBUF + 1:])
        wid = lax.axis_index("s") * 2 + lax.axis_index("c")

        @pl.when(wid < NUM_GATHER_WORKERS)
        def _lists():
            lst = wid // 4
            q = wid % 4
            for i, ref in enumerate((l0, l1, l2, l3, l4, l5, l6)):
                @pl.when(lst == i)
                def _(ref=ref):
                    pltpu.sync_copy(ref, idx_v)

            iota = lax.iota(jnp.int32, LANES)
            base = q * IDS_PER_WORKER
            ivs = []
            for g in range(4):
                gidx = iota + (base + LANES * g)
                if g == 3:
                    gidx = jnp.minimum(gidx, LIST_LEN - 1)
                ivs.append(plsc.load_gather(idx_v, [gidx]))

            cps = [None] * IDS_PER_WORKER
            lanes = [None] * IDS_PER_WORKER

            def fire(k, slot):
                r = ivs[k // LANES][k % LANES]
                jt = pl.multiple_of((r >> 7) << 7, 128)
                lanes[k] = r & 127
                cps[k] = pltpu.async_copy(
                    tt_hbm.at[:, pl.ds(jt, 128)], bufs[slot], sems[slot])

            for k in range(NBUF):
                fire(k, k)
            z = jnp.zeros((LANES,), jnp.float32)
            accs = [z, z, z, z]
            iotas = [iota + LANES * c for c in range(4)]
            for k in range(IDS_PER_WORKER):
                slot = k % NBUF
                cps[k].wait()
                colidx = jnp.full((LANES,), lanes[k], jnp.int32)
                for c in range(4):
                    rows = plsc.load_gather(bufs[slot], [iotas[c], colidx])
                    accs[c] = accs[c] + rows
                if k + NBUF < IDS_PER_WORKER:
                    fire(k + NBUF, slot)
            for c in range(4):
                outv[0, pl.ds(LANES * c, LANES)] = accs[c]
            pltpu.sync_copy(outv, out_hbm.at[NUM_LISTS * q + lst])

    return enc(*id_lists, tt)


def _tc_combine_body(svec_ref, x_ref, w1_ref, b1_ref, w2_ref, b2_ref, o_ref):
    scale = jnp.float32(1.0 / LIST_LEN)
    o_ref[pl.ds(0, NUM_LISTS), :] = (
        x_ref[pl.ds(0, NUM_LISTS), 0, :]
        + x_ref[pl.ds(NUM_LISTS, NUM_LISTS), 0, :]
        + x_ref[pl.ds(2 * NUM_LISTS, NUM_LISTS), 0, :]
        + x_ref[pl.ds(3 * NUM_LISTS, NUM_LISTS), 0, :]) * scale
    w1 = w1_ref[...]
    h = b1_ref[...]
    for k in range(NUM_LISTS):
        h = h + svec_ref[k] * w1[:, k]
    h = jnp.maximum(h, 0.0)
    w2 = w2_ref[...]
    out = b2_ref[...]
    for j in range(32):
        out = out + h[j] * w2[:, j]
    o_ref[pl.ds(NUM_LISTS, 1), :] = out[None, :]


def _tc_combine(svec, partials, W1, b1, W2, b2):
    return pl.pallas_call(
        _tc_combine_body,
        in_specs=[
            pl.BlockSpec(memory_space=pltpu.SMEM),
            pl.BlockSpec(memory_space=pltpu.VMEM),
            pl.BlockSpec(memory_space=pltpu.VMEM),
            pl.BlockSpec(memory_space=pltpu.VMEM),
            pl.BlockSpec(memory_space=pltpu.VMEM),
            pl.BlockSpec(memory_space=pltpu.VMEM),
        ],
        out_shape=jax.ShapeDtypeStruct((NUM_LISTS + 1, DIM), jnp.float32),
    )(svec, partials, W1, b1, W2, b2)


def kernel(self_main_ids, self_field_ids, self_graveyard_ids, self_banish_ids,
           opp_fields_ids, opp_graveyard_ids, opp_banish_ids,
           self_banish_verso, opp_banish_verso, opp_fields_verso_card,
           phase_id, lp, adv_lp, opp_main,
           table, W1, b1, W2, b2):
    id_lists = [self_main_ids.astype(jnp.int32),
                self_field_ids.astype(jnp.int32),
                opp_fields_ids.astype(jnp.int32),
                self_graveyard_ids.astype(jnp.int32),
                self_banish_ids.astype(jnp.int32),
                opp_graveyard_ids.astype(jnp.int32),
                opp_banish_ids.astype(jnp.int32)]
    svec = jnp.stack([phase_id, opp_main, lp, adv_lp,
                      opp_fields_verso_card, self_banish_verso,
                      opp_banish_verso]).astype(jnp.float32)

    partials = _sc_gather(id_lists, table.T)
    out = _tc_combine(svec, partials, W1, b1, W2, b2)
    return out.reshape(NUM_LISTS * DIM + DIM)


# speedup vs baseline: 14.2075x; 1.0416x over previous
import functools

import jax
import jax.numpy as jnp
from jax import lax
from jax.experimental import pallas as pl
from jax.experimental.pallas import tpu as pltpu
from jax.experimental.pallas import tpu_sc as plsc

DIM = 64
LIST_LEN = 200
NUM_LISTS = 7
LANES = 16
IDS_PER_WORKER = 50
NUM_GATHER_WORKERS = 28
NBUF = 10


def _sc_gather(id_lists, tt):
    mesh = plsc.VectorSubcoreMesh(
        core_axis_name="c", subcore_axis_name="s",
        num_cores=2, num_subcores=16)

    @functools.partial(
        pl.kernel,
        mesh=mesh,
        out_type=jax.ShapeDtypeStruct((NUM_GATHER_WORKERS, 1, DIM),
                                      jnp.float32),
        compiler_params=pltpu.CompilerParams(needs_layout_passes=False),
        scratch_types=[
            pltpu.VMEM((LIST_LEN,), jnp.int32),
            *[pltpu.VMEM((DIM, 128), jnp.float32) for _ in range(NBUF)],
            pltpu.VMEM((1, DIM), jnp.float32),
            *[pltpu.SemaphoreType.DMA for _ in range(NBUF)],
        ],
    )
    def enc(l0, l1, l2, l3, l4, l5, l6, tt_hbm, out_hbm, idx_v, *rest):
        bufs = list(rest[:NBUF])
        outv = rest[NBUF]
        sems = list(rest[NBUF + 1:])
        wid = lax.axis_index("s") * 2 + lax.axis_index("c")

        @pl.when(wid < NUM_GATHER_WORKERS)
        def _lists():
            lst = wid // 4
            q = wid % 4
            for i, ref in enumerate((l0, l1, l2, l3, l4, l5, l6)):
                @pl.when(lst == i)
                def _(ref=ref):
                    pltpu.sync_copy(ref, idx_v)

            iota = lax.iota(jnp.int32, LANES)
            base = q * IDS_PER_WORKER
            ivs = []
            for g in range(4):
                gidx = iota + (base + LANES * g)
                if g == 3:
                    gidx = jnp.minimum(gidx, LIST_LEN - 1)
                ivs.append(plsc.load_gather(idx_v, [gidx]))

            cps = [None] * IDS_PER_WORKER
            lanes = [None] * IDS_PER_WORKER

            def fire(k, slot):
                r = ivs[k // LANES][k % LANES]
                jt = pl.multiple_of((r >> 7) << 7, 128)
                lanes[k] = r & 127
                cps[k] = pltpu.async_copy(
                    tt_hbm.at[:, pl.ds(jt, 128)], bufs[slot], sems[slot])

            for k in range(NBUF):
                fire(k, k)
            z = jnp.zeros((LANES,), jnp.float32)
            accs = [z, z, z, z]
            iotas = [iota + LANES * c for c in range(4)]
            for k in range(IDS_PER_WORKER):
                slot = k % NBUF
                cps[k].wait()
                colidx = jnp.full((LANES,), lanes[k], jnp.int32)
                for c in range(4):
                    rows = plsc.load_gather(bufs[slot], [iotas[c], colidx])
                    accs[c] = accs[c] + rows
                if k + NBUF < IDS_PER_WORKER:
                    fire(k + NBUF, slot)
            for c in range(4):
                outv[0, pl.ds(LANES * c, LANES)] = accs[c]
            pltpu.sync_copy(outv, out_hbm.at[NUM_LISTS * q + lst])

    return enc(*id_lists, tt)


def _tc_combine_body(svec_ref, x_ref, w1p_ref, w2t_ref, b2_ref, o_ref):
    scale = jnp.float32(1.0 / LIST_LEN)
    sums = (x_ref[pl.ds(0, NUM_LISTS), 0, :]
            + x_ref[pl.ds(NUM_LISTS, NUM_LISTS), 0, :]
            + x_ref[pl.ds(2 * NUM_LISTS, NUM_LISTS), 0, :]
            + x_ref[pl.ds(3 * NUM_LISTS, NUM_LISTS), 0, :]) * scale
    o_ref[pl.ds(0, NUM_LISTS), :] = sums
    s = jnp.stack([svec_ref[k] for k in range(8)])[None, :]
    h = jnp.maximum(
        jnp.dot(s, w1p_ref[...], precision=lax.Precision.HIGHEST), 0.0)
    out = jnp.dot(h, w2t_ref[...], precision=lax.Precision.HIGHEST)
    o_ref[pl.ds(NUM_LISTS, 1), :] = out + b2_ref[...][None, :]


def _tc_combine(svec, partials, w1p, w2t, b2):
    return pl.pallas_call(
        _tc_combine_body,
        in_specs=[
            pl.BlockSpec(memory_space=pltpu.SMEM),
            pl.BlockSpec(memory_space=pltpu.VMEM),
            pl.BlockSpec(memory_space=pltpu.VMEM),
            pl.BlockSpec(memory_space=pltpu.VMEM),
            pl.BlockSpec(memory_space=pltpu.VMEM),
        ],
        out_shape=jax.ShapeDtypeStruct((NUM_LISTS + 1, DIM), jnp.float32),
    )(svec, partials, w1p, w2t, b2)


def kernel(self_main_ids, self_field_ids, self_graveyard_ids, self_banish_ids,
           opp_fields_ids, opp_graveyard_ids, opp_banish_ids,
           self_banish_verso, opp_banish_verso, opp_fields_verso_card,
           phase_id, lp, adv_lp, opp_main,
           table, W1, b1, W2, b2):
    id_lists = [self_main_ids.astype(jnp.int32),
                self_field_ids.astype(jnp.int32),
                opp_fields_ids.astype(jnp.int32),
                self_graveyard_ids.astype(jnp.int32),
                self_banish_ids.astype(jnp.int32),
                opp_graveyard_ids.astype(jnp.int32),
                opp_banish_ids.astype(jnp.int32)]
    svec = jnp.stack([phase_id, opp_main, lp, adv_lp,
                      opp_fields_verso_card, self_banish_verso,
                      opp_banish_verso, 1]).astype(jnp.float32)
    w1p = jnp.concatenate([W1.T, b1[None, :]], axis=0)
    w2t = W2.T

    partials = _sc_gather(id_lists, table.T)
    out = _tc_combine(svec, partials, w1p, w2t, b2)
    return out.reshape(NUM_LISTS * DIM + DIM)
